# baseline (speedup 1.0000x reference)
import functools

import jax
import jax.numpy as jnp
from jax import lax
from jax.experimental import pallas as pl
from jax.experimental.pallas import tpu as pltpu
from jax.experimental.pallas import tpu_sc as plsc

VOCAB = 1000000
D_MODEL = 64
ROW = 128
BATCH = 4096
HIST = 200
B = BATCH * HIST

NUM_CORES = 2
NUM_SUBCORES = 16
NW = NUM_CORES * NUM_SUBCORES
BPW = B // NW
NBUF = 4
CHUNK = 200
NCHUNK = BPW // CHUNK
NGROUP = NCHUNK // NBUF

_mesh = plsc.VectorSubcoreMesh(core_axis_name="c", subcore_axis_name="s")


@functools.partial(
    pl.kernel,
    mesh=_mesh,
    out_type=jax.ShapeDtypeStruct((B, ROW), jnp.float32),
    scratch_types=[
        pltpu.VMEM((BPW,), jnp.int32),
        pltpu.VMEM((NBUF, CHUNK, ROW), jnp.float32),
        pltpu.SemaphoreType.DMA,
        pltpu.SemaphoreType.DMA,
        pltpu.SemaphoreType.DMA,
        pltpu.SemaphoreType.DMA,
        pltpu.SemaphoreType.DMA,
        pltpu.SemaphoreType.DMA,
        pltpu.SemaphoreType.DMA,
        pltpu.SemaphoreType.DMA,
    ],
    compiler_params=pltpu.CompilerParams(use_tc_tiling_on_sc=False),
)
def _gather_kernel(idx_hbm, table_hbm, out_hbm, idx_v, rows, sg0, sg1, sg2,
                   sg3, ss0, ss1, ss2, ss3):
    sem_g = [sg0, sg1, sg2, sg3]
    sem_s = [ss0, ss1, ss2, ss3]
    wid = lax.axis_index("s") * NUM_CORES + lax.axis_index("c")
    base = wid * BPW

    pltpu.sync_copy(idx_hbm.at[pl.ds(base, BPW)], idx_v)

    def gather_desc(chunk, b):
        idx_sl = idx_v.at[pl.ds(chunk * CHUNK, CHUNK)]
        return pltpu.make_async_copy(table_hbm.at[idx_sl], rows.at[b],
                                     sem_g[b])

    def store_desc(chunk, b):
        off = base + chunk * CHUNK
        return pltpu.make_async_copy(rows.at[b], out_hbm.at[pl.ds(off, CHUNK)],
                                     sem_s[b])

    for b in range(NBUF):
        gather_desc(b, b).start()

    def body(j, carry):
        for b in range(NBUF):
            i = j * NBUF + b
            gather_desc(i, b).wait()
            store_desc(i, b).start()
            store_desc(i, b).wait()
            gather_desc(i + NBUF, b).start()
        return carry

    lax.fori_loop(0, NGROUP - 1, body, 0)

    last = (NGROUP - 1) * NBUF
    for b in range(NBUF):
        gather_desc(last + b, b).wait()
        store_desc(last + b, b).start()
    for b in range(NBUF):
        store_desc(last + b, b).wait()


def kernel(x, table):
    idx = x.reshape(-1).astype(jnp.int32)
    table_p = jnp.pad(table, ((0, 0), (0, ROW - D_MODEL)))
    out2 = _gather_kernel(idx, table_p)
    out = lax.slice(out2, (0, 0), (B, D_MODEL))
    return out.reshape(x.shape + (D_MODEL,))


# speedup vs baseline: 1.0913x; 1.0913x over previous
import functools

import jax
import jax.numpy as jnp
from jax import lax
from jax.experimental import pallas as pl
from jax.experimental.pallas import tpu as pltpu
from jax.experimental.pallas import tpu_sc as plsc

VOCAB = 1000000
D_MODEL = 64
ROW = 128
BATCH = 4096
HIST = 200
B = BATCH * HIST

NUM_CORES = 2
NUM_SUBCORES = 16
NW = NUM_CORES * NUM_SUBCORES
BPW = B // NW
NBUF = 4
CHUNK = 400
NCHUNK = BPW // CHUNK
NGROUP = NCHUNK // NBUF

_mesh = plsc.VectorSubcoreMesh(core_axis_name="c", subcore_axis_name="s")


@functools.partial(
    pl.kernel,
    mesh=_mesh,
    out_type=jax.ShapeDtypeStruct((B, ROW), jnp.float32),
    scratch_types=[
        pltpu.VMEM((BPW,), jnp.int32),
        pltpu.VMEM((NBUF, CHUNK, D_MODEL), jnp.float32),
        pltpu.SemaphoreType.DMA,
        pltpu.SemaphoreType.DMA,
        pltpu.SemaphoreType.DMA,
        pltpu.SemaphoreType.DMA,
        pltpu.SemaphoreType.DMA,
        pltpu.SemaphoreType.DMA,
        pltpu.SemaphoreType.DMA,
        pltpu.SemaphoreType.DMA,
    ],
    compiler_params=pltpu.CompilerParams(use_tc_tiling_on_sc=False),
)
def _gather_kernel(idx_hbm, table_hbm, out_hbm, idx_v, rows, sg0, sg1, sg2,
                   sg3, ss0, ss1, ss2, ss3):
    sem_g = [sg0, sg1, sg2, sg3]
    sem_s = [ss0, ss1, ss2, ss3]
    wid = lax.axis_index("s") * NUM_CORES + lax.axis_index("c")
    base = wid * BPW

    pltpu.sync_copy(idx_hbm.at[pl.ds(base, BPW)], idx_v)

    def gather_desc(chunk, b):
        idx_sl = idx_v.at[pl.ds(chunk * CHUNK, CHUNK)]
        return pltpu.make_async_copy(table_hbm.at[idx_sl], rows.at[b],
                                     sem_g[b])

    def store_desc(chunk, b):
        off = base + chunk * CHUNK
        dst = out_hbm.at[pl.ds(off, CHUNK), pl.ds(0, D_MODEL)]
        return pltpu.make_async_copy(rows.at[b], dst, sem_s[b])

    for b in range(NBUF):
        gather_desc(b, b).start()

    def body(j, carry):
        for b in range(NBUF):
            i = j * NBUF + b
            gather_desc(i, b).wait()
            store_desc(i, b).start()
            store_desc(i, b).wait()
            gather_desc(i + NBUF, b).start()
        return carry

    lax.fori_loop(0, NGROUP - 1, body, 0)

    last = (NGROUP - 1) * NBUF
    for b in range(NBUF):
        gather_desc(last + b, b).wait()
        store_desc(last + b, b).start()
    for b in range(NBUF):
        store_desc(last + b, b).wait()


def kernel(x, table):
    idx = x.reshape(-1).astype(jnp.int32)
    t2 = lax.optimization_barrier(table.reshape(VOCAB // 2, 2 * D_MODEL))
    table_lin = t2.reshape(VOCAB, D_MODEL)
    out2 = _gather_kernel(idx, table_lin)
    out = lax.slice(out2, (0, 0), (B, D_MODEL))
    return out.reshape(x.shape + (D_MODEL,))
